# baseline (speedup 1.0000x reference)
import jax
import jax.numpy as jnp
from jax import lax
from jax.experimental import pallas as pl
from jax.experimental.pallas import tpu as pltpu
from jax.experimental.pallas import tpu_sc as plsc

_B = 16384
_NW = 32
_RW = _B // _NW
_G = 32
_NG = _RW // _G
_XW = 128


def _sc_gather_kernel(
    sp_h, mv0_h, mv1_h, mv2_h, mv3_h, ab_h, it_h, t1_h,
    pok_h, mv_h, ab_tab_h, it_tab_h, ty_h,
    x_o,
    pok_v, mv_v, ab_v, it_v, ty_v,
    sp_i, mv0_i, mv1_i, mv2_i, mv3_i, ab_i, it_i, t1_i,
    xball, isem, wsem,
):
  wid = lax.axis_index("c") * 16 + lax.axis_index("s")
  base = wid * _RW
  bsl = pl.ds(base, _RW)

  idx_bufs = (sp_i, mv0_i, mv1_i, mv2_i, mv3_i, ab_i, it_i, t1_i)
  idx_hbm = (sp_h, mv0_h, mv1_h, mv2_h, mv3_h, ab_h, it_h, t1_h)

  ih = [pltpu.async_copy(h, v, isem) for h, v in
        ((pok_h, pok_v), (mv_h, mv_v), (ab_tab_h, ab_v), (it_tab_h, it_v),
         (ty_h, ty_v))]
  ih += [pltpu.async_copy(h.at[bsl], buf, isem)
         for h, buf in zip(idx_hbm, idx_bufs)]
  for h in ih:
    h.wait()

  mv_is = (mv0_i, mv1_i, mv2_i, mv3_i)

  def remap_body(i, _):
    s2 = pl.ds(i * 16, 16)
    for j in range(4):
      v = mv_is[j][s2]
      mv_is[j][s2] = jnp.where(v != 0, v, 920)
    return 0

  lax.fori_loop(0, _RW // 16, remap_body, 0)

  gsz = _G * _XW
  iota16 = lax.iota(jnp.int32, 16)

  def grp_body(g, _):
    obase = (g % 2) * gsz

    @pl.when(g >= 2)
    def _reclaim():
      pltpu.make_async_copy(
          xball.at[pl.ds(0, gsz)],
          x_o.at[pl.ds(base * _XW, gsz)], wsem).wait()

    for sub in range(_G // 16):
      ssl = pl.ds((g * (_G // 16) + sub) * 16, 16)
      sib = sp_i[ssl] * 48
      i0b = mv0_i[ssl] * 32
      i1b = mv1_i[ssl] * 32
      i2b = mv2_i[ssl] * 32
      i3b = mv3_i[ssl] * 32
      abb = ab_i[ssl] * 16
      itb = it_i[ssl] * 16
      t1b = t1_i[ssl] * 16
      rb = obase + sub * 16 * _XW + iota16 * _XW
      for c in range(48):
        plsc.store_scatter(xball, [rb + c],
                           plsc.load_gather(pok_v, [sib + c]))
      for c in range(32):
        acc = (plsc.load_gather(mv_v, [i0b + c])
               + plsc.load_gather(mv_v, [i1b + c])
               + plsc.load_gather(mv_v, [i2b + c])
               + plsc.load_gather(mv_v, [i3b + c]))
        plsc.store_scatter(xball, [rb + (48 + c)], acc)
      for c in range(16):
        plsc.store_scatter(xball, [rb + (80 + c)],
                           plsc.load_gather(ab_v, [abb + c]))
        plsc.store_scatter(xball, [rb + (96 + c)],
                           plsc.load_gather(it_v, [itb + c]))
        plsc.store_scatter(xball, [rb + (112 + c)],
                           plsc.load_gather(ty_v, [t1b + c]))
    pltpu.async_copy(
        xball.at[pl.ds(obase, gsz)],
        x_o.at[pl.ds((base + g * _G) * _XW, gsz)], wsem)
    return 0

  lax.fori_loop(0, _NG, grp_body, 0)
  for _ in range(2):
    pltpu.make_async_copy(
        xball.at[pl.ds(0, gsz)],
        x_o.at[pl.ds(base * _XW, gsz)], wsem).wait()


def _make_sc_gather():
  f32 = jnp.float32
  i32 = jnp.int32
  out_type = [
      jax.ShapeDtypeStruct((_B * _XW,), f32),
  ]
  scratch = [
      pltpu.VMEM((1025 * 48,), f32),
      pltpu.VMEM((921 * 32,), f32),
      pltpu.VMEM((310 * 16,), f32),
      pltpu.VMEM((1200 * 16,), f32),
      pltpu.VMEM((19 * 16,), f32),
      *[pltpu.VMEM((_RW,), i32) for _ in range(8)],
      pltpu.VMEM((2 * _G * _XW,), f32),
      pltpu.SemaphoreType.DMA,
      pltpu.SemaphoreType.DMA,
  ]
  mesh = plsc.VectorSubcoreMesh(core_axis_name="c", subcore_axis_name="s")
  return pl.kernel(
      _sc_gather_kernel, out_type=out_type, mesh=mesh,
      scratch_types=scratch,
      compiler_params=pltpu.CompilerParams(use_tc_tiling_on_sc=False,
                                           needs_layout_passes=False))


_sc_gather = _make_sc_gather()

_BS = 1024


def _tc_mlp_kernel(x, mvi, tyi, mti, ff, tytab, w1a, w1t2, w1mt, w1ff,
                   b1, w2, b2, out):
  f32 = jnp.float32
  xv = x[...]

  mv = mvi[...]
  nz = (mv != 0).astype(f32)
  cnt = nz[:, 0:1] + nz[:, 1:2] + nz[:, 2:3] + nz[:, 3:4]
  rm = 1.0 / jnp.maximum(cnt, 1.0)
  cols128 = lax.broadcasted_iota(jnp.int32, (_BS, _XW), 1)
  xs = xv * jnp.where((cols128 >= 48) & (cols128 < 80), rm, 1.0)

  cols = lax.broadcasted_iota(jnp.int32, (_BS, 32), 1)
  t2 = tyi[...][:, 1:2]
  oh2 = (cols == t2).astype(f32)
  mt = mti[...]
  mtnz = (mt != 0)
  ohsum = jnp.zeros((_BS, 32), f32)
  for j in range(4):
    c = mt[:, j:j + 1]
    ohsum = ohsum + ((cols == c) & (c != 0)).astype(f32)
  ctf = mtnz.astype(f32)
  ct = ctf[:, 0:1] + ctf[:, 1:2] + ctf[:, 2:3] + ctf[:, 3:4]
  ohs = ohsum * (1.0 / jnp.maximum(ct, 1.0))

  p2 = jnp.dot(tytab[...], w1t2[...], preferred_element_type=f32)
  pt = jnp.dot(tytab[...], w1mt[...], preferred_element_type=f32)
  h = (jnp.dot(xs, w1a[...], preferred_element_type=f32)
       + jnp.dot(oh2, p2, preferred_element_type=f32)
       + jnp.dot(ohs, pt, preferred_element_type=f32)
       + jnp.dot(ff[...], w1ff[...], preferred_element_type=f32)
       + b1[...])
  h = jnp.maximum(h, 0.0)
  out[...] = jnp.maximum(
      jnp.dot(h, w2[...], preferred_element_type=f32) + b2[...], 0.0)


def _make_tc_mlp():
  def bspec(cols):
    return pl.BlockSpec((_BS, cols), lambda i: (i, 0))
  in_specs = [
      bspec(_XW),
      bspec(4),
      bspec(2),
      bspec(4),
      bspec(31),
      pl.BlockSpec((32, 16), lambda i: (0, 0)),
      pl.BlockSpec((128, 256), lambda i: (0, 0)),
      pl.BlockSpec((16, 256), lambda i: (0, 0)),
      pl.BlockSpec((16, 256), lambda i: (0, 0)),
      pl.BlockSpec((31, 256), lambda i: (0, 0)),
      pl.BlockSpec((1, 256), lambda i: (0, 0)),
      pl.BlockSpec((256, 128), lambda i: (0, 0)),
      pl.BlockSpec((1, 128), lambda i: (0, 0)),
  ]
  return pl.pallas_call(
      _tc_mlp_kernel,
      grid=(_B // _BS,),
      in_specs=in_specs,
      out_specs=pl.BlockSpec((_BS, 128), lambda i: (i, 0)),
      out_shape=jax.ShapeDtypeStruct((_B, 128), jnp.float32),
  )


_tc_mlp = _make_tc_mlp()


def kernel(species_idx, move_indices, ability_idx, item_idx, type_indices,
           move_type_indices, float_features, pokemon_table, move_table,
           ability_table, item_table, type_table, W1, b1, W2, b2):
  f32 = jnp.float32
  mv_tab = jnp.pad(
      jnp.concatenate([move_table, jnp.zeros((1, 24), f32)], axis=0),
      ((0, 0), (0, 8)))
  ty_pad = jnp.pad(type_table, ((0, 13), (0, 0)))
  w1a = jnp.concatenate([W1[:72], jnp.zeros((8, 256), f32), W1[72:120]],
                        axis=0)
  w1t2 = W1[120:136]
  w1mt = W1[136:152]
  w1ff = W1[152:183]

  (x,) = _sc_gather(
      species_idx,
      move_indices[:, 0], move_indices[:, 1],
      move_indices[:, 2], move_indices[:, 3],
      ability_idx, item_idx, type_indices[:, 0],
      pokemon_table.reshape(-1), mv_tab.reshape(-1),
      ability_table.reshape(-1), item_table.reshape(-1),
      type_table.reshape(-1))

  return _tc_mlp(x.reshape(_B, _XW), move_indices, type_indices,
                 move_type_indices, float_features, ty_pad, w1a, w1t2,
                 w1mt, w1ff, b1.reshape(1, 256), W2, b2.reshape(1, 128))


# speedup vs baseline: 1.6090x; 1.6090x over previous
import jax
import jax.numpy as jnp
from jax import lax
from jax.experimental import pallas as pl
from jax.experimental.pallas import tpu as pltpu
from jax.experimental.pallas import tpu_sc as plsc

_B = 16384
_NW = 32
_RW = _B // _NW
_G = 32
_NG = _RW // _G
_XW = 128


def _sc_gather_kernel(
    sp_h, mv0_h, mv1_h, mv2_h, mv3_h, ab_h, it_h, t1_h,
    pok_h, mv_h, ab_tab_h, it_tab_h, ty_h,
    x_o,
    pok_v, mv_v, ab_v, it_v, ty_v,
    sp_i, mv0_i, mv1_i, mv2_i, mv3_i, ab_i, it_i, t1_i,
    xball, isem, wsem,
):
  wid = lax.axis_index("c") * 16 + lax.axis_index("s")
  base = wid * _RW
  bsl = pl.ds(base, _RW)

  idx_bufs = (sp_i, mv0_i, mv1_i, mv2_i, mv3_i, ab_i, it_i, t1_i)
  idx_hbm = (sp_h, mv0_h, mv1_h, mv2_h, mv3_h, ab_h, it_h, t1_h)

  ih = [pltpu.async_copy(h, v, isem) for h, v in
        ((pok_h, pok_v), (mv_h, mv_v), (ab_tab_h, ab_v), (it_tab_h, it_v),
         (ty_h, ty_v))]
  ih += [pltpu.async_copy(h.at[bsl], buf, isem)
         for h, buf in zip(idx_hbm, idx_bufs)]
  for h in ih:
    h.wait()

  mv_is = (mv0_i, mv1_i, mv2_i, mv3_i)

  def remap_body(i, _):
    s2 = pl.ds(i * 16, 16)
    for j in range(4):
      v = mv_is[j][s2]
      mv_is[j][s2] = jnp.where(v != 0, v, 920)
    return 0

  lax.fori_loop(0, _RW // 16, remap_body, 0)

  gsz = _G * _XW

  def grp_body(g, _):
    obase = (g % 2) * gsz

    @pl.when(g >= 2)
    def _reclaim():
      pltpu.make_async_copy(
          xball.at[pl.ds(0, gsz)],
          x_o.at[pl.ds(base * _XW, gsz)], wsem).wait()

    for sub in range(_G // 16):
      ssl = pl.ds((g * (_G // 16) + sub) * 16, 16)
      sib = sp_i[ssl] * 48
      i0b = mv0_i[ssl] * 32
      i1b = mv1_i[ssl] * 32
      i2b = mv2_i[ssl] * 32
      i3b = mv3_i[ssl] * 32
      abb = ab_i[ssl] * 16
      itb = it_i[ssl] * 16
      t1b = t1_i[ssl] * 16
      for j in range(16):
        o = obase + (sub * 16 + j) * _XW
        si = sib[j]
        for c in range(3):
          xball[pl.ds(o + c * 16, 16)] = pok_v[pl.ds(si + c * 16, 16)]
        i0 = i0b[j]
        i1 = i1b[j]
        i2 = i2b[j]
        i3 = i3b[j]
        for c in range(2):
          acc = (mv_v[pl.ds(i0 + c * 16, 16)] + mv_v[pl.ds(i1 + c * 16, 16)]
                 + mv_v[pl.ds(i2 + c * 16, 16)]
                 + mv_v[pl.ds(i3 + c * 16, 16)])
          xball[pl.ds(o + 48 + c * 16, 16)] = acc
        xball[pl.ds(o + 80, 16)] = ab_v[pl.ds(abb[j], 16)]
        xball[pl.ds(o + 96, 16)] = it_v[pl.ds(itb[j], 16)]
        xball[pl.ds(o + 112, 16)] = ty_v[pl.ds(t1b[j], 16)]
    pltpu.async_copy(
        xball.at[pl.ds(obase, gsz)],
        x_o.at[pl.ds((base + g * _G) * _XW, gsz)], wsem)
    return 0

  lax.fori_loop(0, _NG, grp_body, 0)
  for _ in range(2):
    pltpu.make_async_copy(
        xball.at[pl.ds(0, gsz)],
        x_o.at[pl.ds(base * _XW, gsz)], wsem).wait()


def _make_sc_gather():
  f32 = jnp.float32
  i32 = jnp.int32
  out_type = [
      jax.ShapeDtypeStruct((_B * _XW,), f32),
  ]
  scratch = [
      pltpu.VMEM((1025 * 48,), f32),
      pltpu.VMEM((921 * 32,), f32),
      pltpu.VMEM((310 * 16,), f32),
      pltpu.VMEM((1200 * 16,), f32),
      pltpu.VMEM((19 * 16,), f32),
      *[pltpu.VMEM((_RW,), i32) for _ in range(8)],
      pltpu.VMEM((2 * _G * _XW,), f32),
      pltpu.SemaphoreType.DMA,
      pltpu.SemaphoreType.DMA,
  ]
  mesh = plsc.VectorSubcoreMesh(core_axis_name="c", subcore_axis_name="s")
  return pl.kernel(
      _sc_gather_kernel, out_type=out_type, mesh=mesh,
      scratch_types=scratch,
      compiler_params=pltpu.CompilerParams(use_tc_tiling_on_sc=False,
                                           disable_bounds_checks=True))


_sc_gather = _make_sc_gather()

_BS = 1024


def _tc_mlp_kernel(x, mvi, tyi, mti, ff, tytab, w1a, w1t2, w1mt, w1ff,
                   b1, w2, b2, out):
  f32 = jnp.float32
  xv = x[...]

  mv = mvi[...]
  nz = (mv != 0).astype(f32)
  cnt = nz[:, 0:1] + nz[:, 1:2] + nz[:, 2:3] + nz[:, 3:4]
  rm = 1.0 / jnp.maximum(cnt, 1.0)
  cols128 = lax.broadcasted_iota(jnp.int32, (_BS, _XW), 1)
  xs = xv * jnp.where((cols128 >= 48) & (cols128 < 80), rm, 1.0)

  cols = lax.broadcasted_iota(jnp.int32, (_BS, 32), 1)
  t2 = tyi[...][:, 1:2]
  oh2 = (cols == t2).astype(f32)
  mt = mti[...]
  mtnz = (mt != 0)
  ohsum = jnp.zeros((_BS, 32), f32)
  for j in range(4):
    c = mt[:, j:j + 1]
    ohsum = ohsum + ((cols == c) & (c != 0)).astype(f32)
  ctf = mtnz.astype(f32)
  ct = ctf[:, 0:1] + ctf[:, 1:2] + ctf[:, 2:3] + ctf[:, 3:4]
  ohs = ohsum * (1.0 / jnp.maximum(ct, 1.0))

  p2 = jnp.dot(tytab[...], w1t2[...], preferred_element_type=f32)
  pt = jnp.dot(tytab[...], w1mt[...], preferred_element_type=f32)
  h = (jnp.dot(xs, w1a[...], preferred_element_type=f32)
       + jnp.dot(oh2, p2, preferred_element_type=f32)
       + jnp.dot(ohs, pt, preferred_element_type=f32)
       + jnp.dot(ff[...], w1ff[...], preferred_element_type=f32)
       + b1[...])
  h = jnp.maximum(h, 0.0)
  out[...] = jnp.maximum(
      jnp.dot(h, w2[...], preferred_element_type=f32) + b2[...], 0.0)


def _make_tc_mlp():
  def bspec(cols):
    return pl.BlockSpec((_BS, cols), lambda i: (i, 0))
  in_specs = [
      bspec(_XW),
      bspec(4),
      bspec(2),
      bspec(4),
      bspec(31),
      pl.BlockSpec((32, 16), lambda i: (0, 0)),
      pl.BlockSpec((128, 256), lambda i: (0, 0)),
      pl.BlockSpec((16, 256), lambda i: (0, 0)),
      pl.BlockSpec((16, 256), lambda i: (0, 0)),
      pl.BlockSpec((31, 256), lambda i: (0, 0)),
      pl.BlockSpec((1, 256), lambda i: (0, 0)),
      pl.BlockSpec((256, 128), lambda i: (0, 0)),
      pl.BlockSpec((1, 128), lambda i: (0, 0)),
  ]
  return pl.pallas_call(
      _tc_mlp_kernel,
      grid=(_B // _BS,),
      in_specs=in_specs,
      out_specs=pl.BlockSpec((_BS, 128), lambda i: (i, 0)),
      out_shape=jax.ShapeDtypeStruct((_B, 128), jnp.float32),
  )


_tc_mlp = _make_tc_mlp()


def kernel(species_idx, move_indices, ability_idx, item_idx, type_indices,
           move_type_indices, float_features, pokemon_table, move_table,
           ability_table, item_table, type_table, W1, b1, W2, b2):
  f32 = jnp.float32
  mv_tab = jnp.pad(
      jnp.concatenate([move_table, jnp.zeros((1, 24), f32)], axis=0),
      ((0, 0), (0, 8)))
  ty_pad = jnp.pad(type_table, ((0, 13), (0, 0)))
  w1a = jnp.concatenate([W1[:72], jnp.zeros((8, 256), f32), W1[72:120]],
                        axis=0)
  w1t2 = W1[120:136]
  w1mt = W1[136:152]
  w1ff = W1[152:183]

  (x,) = _sc_gather(
      species_idx,
      move_indices[:, 0], move_indices[:, 1],
      move_indices[:, 2], move_indices[:, 3],
      ability_idx, item_idx, type_indices[:, 0],
      pokemon_table.reshape(-1), mv_tab.reshape(-1),
      ability_table.reshape(-1), item_table.reshape(-1),
      type_table.reshape(-1))

  return _tc_mlp(x.reshape(_B, _XW), move_indices, type_indices,
                 move_type_indices, float_features, ty_pad, w1a, w1t2,
                 w1mt, w1ff, b1.reshape(1, 256), W2, b2.reshape(1, 128))
